# Initial kernel scaffold; baseline (speedup 1.0000x reference)
import jax
import jax.numpy as jnp
from jax.experimental import pallas as pl


def kernel(x, edge_index, W1l, b1l, W1r, gamma, beta, W2l, b2l, W2r):
    raise NotImplementedError("write your pallas kernel here")


# speedup vs baseline: 7.8843x; 7.8843x over previous
import functools

import jax
import jax.numpy as jnp
from jax import lax
from jax.experimental import pallas as pl
from jax.experimental.pallas import tpu as pltpu
from jax.experimental.pallas import tpu_sc as plsc

N = 10000
ACC_N = 10240
IN_DIM = 256
HID = 64
OUT = 16
D1 = HID + 16
D2 = OUT

NUM_CORES = 2
NUM_TILES = 16
NUM_WORKERS = NUM_CORES * NUM_TILES
CHUNK = 128
ROWS_PER_TILE = ACC_N // NUM_TILES


def _mm1_body(x_ref, wl_ref, wr_ref, b_ref, ya_ref, xr_ref):
    x = x_ref[...]
    xl = jnp.dot(x, wl_ref[...], preferred_element_type=jnp.float32,
                 precision=lax.Precision.HIGHEST)
    blk = xl.shape[0]
    ya_ref[...] = jnp.concatenate(
        [xl, jnp.ones((blk, D1 - HID), jnp.float32)], axis=1)
    xr_ref[...] = jnp.dot(x, wr_ref[...], preferred_element_type=jnp.float32,
                          precision=lax.Precision.HIGHEST) + b_ref[...]


def _mm1(x, wlT, wrT, b1l2d):
    blk = 2000
    return pl.pallas_call(
        _mm1_body,
        grid=(N // blk,),
        in_specs=[
            pl.BlockSpec((blk, IN_DIM), lambda i: (i, 0)),
            pl.BlockSpec((IN_DIM, HID), lambda i: (0, 0)),
            pl.BlockSpec((IN_DIM, HID), lambda i: (0, 0)),
            pl.BlockSpec((1, HID), lambda i: (0, 0)),
        ],
        out_specs=[
            pl.BlockSpec((blk, D1), lambda i: (i, 0)),
            pl.BlockSpec((blk, HID), lambda i: (i, 0)),
        ],
        out_shape=[
            jax.ShapeDtypeStruct((N, D1), jnp.float32),
            jax.ShapeDtypeStruct((N, HID), jnp.float32),
        ],
    )(x, wlT, wrT, b1l2d)


def _make_scatter(d, chunks_per_tile):
    mesh = plsc.VectorSubcoreMesh(core_axis_name="c", subcore_axis_name="s")

    @functools.partial(
        pl.kernel,
        mesh=mesh,
        compiler_params=pltpu.CompilerParams(use_tc_tiling_on_sc=False),
        out_type=jax.ShapeDtypeStruct((NUM_CORES * ACC_N, d), jnp.float32),
        scratch_types=[
            pltpu.VMEM((CHUNK,), jnp.int32),
            pltpu.VMEM((CHUNK,), jnp.int32),
            pltpu.VMEM((CHUNK, d), jnp.float32),
            pltpu.VMEM_SHARED((ACC_N, d), jnp.float32),
            pltpu.SemaphoreType.DMA,
        ],
    )
    def scatter(table_hbm, src_hbm, dst_hbm, zeros_hbm, out_hbm,
                src_v, dst_v, rows_v, acc_sh, sem):
        cid = lax.axis_index("c")
        sid = lax.axis_index("s")
        wid = cid * NUM_TILES + sid

        pltpu.sync_copy(zeros_hbm, acc_sh.at[pl.ds(sid * ROWS_PER_TILE,
                                                   ROWS_PER_TILE)])
        plsc.subcore_barrier()

        def body(i, carry):
            base = (wid * chunks_per_tile + i) * CHUNK
            pltpu.sync_copy(src_hbm.at[pl.ds(base, CHUNK)], src_v)
            pltpu.sync_copy(dst_hbm.at[pl.ds(base, CHUNK)], dst_v)
            pltpu.async_copy(table_hbm.at[src_v], rows_v, sem).wait()
            pltpu.sync_copy(rows_v, acc_sh.at[dst_v], add=True)
            return carry

        lax.fori_loop(0, chunks_per_tile, body, 0)
        plsc.subcore_barrier()

        pltpu.sync_copy(
            acc_sh.at[pl.ds(sid * ROWS_PER_TILE, ROWS_PER_TILE)],
            out_hbm.at[pl.ds(cid * ACC_N + sid * ROWS_PER_TILE,
                             ROWS_PER_TILE)])

    return scatter


def _mid_body(acc_ref, ya_ref, xr_ref, g_ref, b_ref, w2_ref, b2_ref,
              z1_ref, zr_ref, rcnt_ref):
    a = acc_ref[...]
    s = a[:N] + a[ACC_N:ACC_N + N]
    cnt = s[:, HID:HID + 1] + 1.0
    rcnt = 1.0 / cnt
    y1 = ya_ref[...][:, :HID]
    h = (s[:, :HID] + y1) * rcnt + xr_ref[...]
    mean = jnp.mean(h, axis=0, keepdims=True)
    var = jnp.mean((h - mean) ** 2, axis=0, keepdims=True)
    hn = (h - mean) * lax.rsqrt(var + 1e-5) * g_ref[...] + b_ref[...]
    hn = jnp.maximum(hn, 0.0)
    z = jnp.dot(hn, w2_ref[...], preferred_element_type=jnp.float32,
                precision=lax.Precision.HIGHEST)
    z1_ref[...] = z[:, :OUT]
    zr_ref[...] = z[:, OUT:] + b2_ref[...]
    rcnt_ref[...] = jnp.broadcast_to(rcnt, (N, OUT))


def _mid(acc, ya, xr, gamma2d, beta2d, w2cat, b2l2d):
    return pl.pallas_call(
        _mid_body,
        out_shape=[
            jax.ShapeDtypeStruct((N, OUT), jnp.float32),
            jax.ShapeDtypeStruct((N, OUT), jnp.float32),
            jax.ShapeDtypeStruct((N, OUT), jnp.float32),
        ],
    )(acc, ya, xr, gamma2d, beta2d, w2cat, b2l2d)


def _fin_body(acc2_ref, z1_ref, rcnt_ref, zr_ref, out_ref):
    a = acc2_ref[...]
    s = a[:N] + a[ACC_N:ACC_N + N]
    out_ref[...] = (s + z1_ref[...]) * rcnt_ref[...] + zr_ref[...]


def _fin(acc2, z1, rcnt, zr):
    return pl.pallas_call(
        _fin_body,
        out_shape=jax.ShapeDtypeStruct((N, OUT), jnp.float32),
    )(acc2, z1, rcnt, zr)


def kernel(x, edge_index, W1l, b1l, W1r, gamma, beta, W2l, b2l, W2r):
    e = edge_index.shape[1]
    epad = ((e + NUM_WORKERS * CHUNK - 1) // (NUM_WORKERS * CHUNK)
            * (NUM_WORKERS * CHUNK))
    chunks_per_tile = epad // (NUM_WORKERS * CHUNK)

    src_p = jnp.concatenate(
        [edge_index[0], jnp.zeros((epad - e,), jnp.int32)])
    dst_p = jnp.concatenate(
        [edge_index[1], jnp.full((epad - e,), N, jnp.int32)])
    zeros1 = jnp.zeros((ROWS_PER_TILE, D1), jnp.float32)
    zeros2 = jnp.zeros((ROWS_PER_TILE, D2), jnp.float32)

    ya, xr = _mm1(x, W1l.T, W1r.T, b1l[None, :])
    acc = _make_scatter(D1, chunks_per_tile)(ya, src_p, dst_p, zeros1)
    w2cat = jnp.concatenate([W2l.T, W2r.T], axis=1)
    z1, zr, rcnt = _mid(acc, ya, xr, gamma[None, :], beta[None, :],
                        w2cat, b2l[None, :])
    acc2 = _make_scatter(D2, chunks_per_tile)(z1, src_p, dst_p, zeros2)
    return _fin(acc2, z1, rcnt, zr)


# speedup vs baseline: 10.4769x; 1.3288x over previous
import functools

import jax
import jax.numpy as jnp
from jax import lax
from jax.experimental import pallas as pl
from jax.experimental.pallas import tpu as pltpu
from jax.experimental.pallas import tpu_sc as plsc

N = 10000
ACC_N = 10240
IN_DIM = 256
HID = 64
OUT = 16
D1 = HID + 16
D2 = OUT

NUM_CORES = 2
NUM_TILES = 16
NUM_WORKERS = NUM_CORES * NUM_TILES
CHUNK = 128
ROWS_PER_TILE = ACC_N // NUM_TILES


def _mm1_body(x_ref, wl_ref, wr_ref, b_ref, ya_ref, xr_ref):
    x = x_ref[...]
    xl = jnp.dot(x, wl_ref[...], preferred_element_type=jnp.float32,
                 precision=lax.Precision.HIGHEST)
    blk = xl.shape[0]
    ya_ref[...] = jnp.concatenate(
        [xl, jnp.ones((blk, D1 - HID), jnp.float32)], axis=1)
    xr_ref[...] = jnp.dot(x, wr_ref[...], preferred_element_type=jnp.float32,
                          precision=lax.Precision.HIGHEST) + b_ref[...]


def _mm1(x, wlT, wrT, b1l2d):
    blk = 2000
    return pl.pallas_call(
        _mm1_body,
        grid=(N // blk,),
        in_specs=[
            pl.BlockSpec((blk, IN_DIM), lambda i: (i, 0)),
            pl.BlockSpec((IN_DIM, HID), lambda i: (0, 0)),
            pl.BlockSpec((IN_DIM, HID), lambda i: (0, 0)),
            pl.BlockSpec((1, HID), lambda i: (0, 0)),
        ],
        out_specs=[
            pl.BlockSpec((blk, D1), lambda i: (i, 0)),
            pl.BlockSpec((blk, HID), lambda i: (i, 0)),
        ],
        out_shape=[
            jax.ShapeDtypeStruct((N, D1), jnp.float32),
            jax.ShapeDtypeStruct((N, HID), jnp.float32),
        ],
    )(x, wlT, wrT, b1l2d)


NBUF = 8


def _make_scatter(d, chunks_per_tile):
    mesh = plsc.VectorSubcoreMesh(core_axis_name="c", subcore_axis_name="s")
    nbuf = 4 if d > 32 else NBUF

    @functools.partial(
        pl.kernel,
        mesh=mesh,
        compiler_params=pltpu.CompilerParams(use_tc_tiling_on_sc=False),
        out_type=jax.ShapeDtypeStruct((NUM_CORES * ACC_N, d), jnp.float32),
        scratch_types=[
            pltpu.VMEM((chunks_per_tile, CHUNK), jnp.int32),
            pltpu.VMEM((chunks_per_tile, CHUNK), jnp.int32),
            pltpu.VMEM((nbuf, CHUNK, d), jnp.float32),
            pltpu.VMEM_SHARED((ACC_N, d), jnp.float32),
            pltpu.SemaphoreType.DMA,
            pltpu.SemaphoreType.DMA,
        ],
    )
    def scatter(table_hbm, src_hbm, dst_hbm, zeros_hbm, out_hbm,
                src_all, dst_all, rows, acc_sh, gsem, ssem):
        cid = lax.axis_index("c")
        sid = lax.axis_index("s")
        wid = cid * NUM_TILES + sid

        pltpu.sync_copy(zeros_hbm, acc_sh.at[pl.ds(sid * ROWS_PER_TILE,
                                                   ROWS_PER_TILE)])
        pltpu.sync_copy(src_hbm.at[pl.ds(wid * chunks_per_tile,
                                         chunks_per_tile)], src_all)
        pltpu.sync_copy(dst_hbm.at[pl.ds(wid * chunks_per_tile,
                                         chunks_per_tile)], dst_all)
        plsc.subcore_barrier()

        def outer(g, carry):
            base = g * nbuf
            gcps = [
                pltpu.async_copy(table_hbm.at[src_all.at[base + b]],
                                 rows.at[b], gsem)
                for b in range(nbuf)
            ]
            scps = []
            for b in range(nbuf):
                gcps[b].wait()
                scps.append(
                    pltpu.async_copy(rows.at[b],
                                     acc_sh.at[dst_all.at[base + b]],
                                     ssem, add=True))
            for cp in scps:
                cp.wait()
            return carry

        lax.fori_loop(0, chunks_per_tile // nbuf, outer, 0)
        plsc.subcore_barrier()

        pltpu.sync_copy(
            acc_sh.at[pl.ds(sid * ROWS_PER_TILE, ROWS_PER_TILE)],
            out_hbm.at[pl.ds(cid * ACC_N + sid * ROWS_PER_TILE,
                             ROWS_PER_TILE)])

    return scatter


def _mid_body(acc_ref, ya_ref, xr_ref, g_ref, b_ref, w2_ref, b2_ref,
              z1_ref, zr_ref, rcnt_ref):
    a = acc_ref[...]
    s = a[:N] + a[ACC_N:ACC_N + N]
    cnt = s[:, HID:HID + 1] + 1.0
    rcnt = 1.0 / cnt
    y1 = ya_ref[...][:, :HID]
    h = (s[:, :HID] + y1) * rcnt + xr_ref[...]
    mean = jnp.mean(h, axis=0, keepdims=True)
    var = jnp.mean((h - mean) ** 2, axis=0, keepdims=True)
    hn = (h - mean) * lax.rsqrt(var + 1e-5) * g_ref[...] + b_ref[...]
    hn = jnp.maximum(hn, 0.0)
    z = jnp.dot(hn, w2_ref[...], preferred_element_type=jnp.float32,
                precision=lax.Precision.HIGHEST)
    z1_ref[...] = z[:, :OUT]
    zr_ref[...] = z[:, OUT:] + b2_ref[...]
    rcnt_ref[...] = jnp.broadcast_to(rcnt, (N, OUT))


def _mid(acc, ya, xr, gamma2d, beta2d, w2cat, b2l2d):
    return pl.pallas_call(
        _mid_body,
        out_shape=[
            jax.ShapeDtypeStruct((N, OUT), jnp.float32),
            jax.ShapeDtypeStruct((N, OUT), jnp.float32),
            jax.ShapeDtypeStruct((N, OUT), jnp.float32),
        ],
    )(acc, ya, xr, gamma2d, beta2d, w2cat, b2l2d)


def _fin_body(acc2_ref, z1_ref, rcnt_ref, zr_ref, out_ref):
    a = acc2_ref[...]
    s = a[:N] + a[ACC_N:ACC_N + N]
    out_ref[...] = (s + z1_ref[...]) * rcnt_ref[...] + zr_ref[...]


def _fin(acc2, z1, rcnt, zr):
    return pl.pallas_call(
        _fin_body,
        out_shape=jax.ShapeDtypeStruct((N, OUT), jnp.float32),
    )(acc2, z1, rcnt, zr)


def kernel(x, edge_index, W1l, b1l, W1r, gamma, beta, W2l, b2l, W2r):
    e = edge_index.shape[1]
    quantum = NUM_WORKERS * CHUNK * NBUF
    epad = (e + quantum - 1) // quantum * quantum
    chunks_per_tile = epad // (NUM_WORKERS * CHUNK)

    src_p = jnp.concatenate(
        [edge_index[0], jnp.zeros((epad - e,), jnp.int32)]
    ).reshape(epad // CHUNK, CHUNK)
    dst_p = jnp.concatenate(
        [edge_index[1], jnp.full((epad - e,), N, jnp.int32)]
    ).reshape(epad // CHUNK, CHUNK)
    zeros1 = jnp.zeros((ROWS_PER_TILE, D1), jnp.float32)
    zeros2 = jnp.zeros((ROWS_PER_TILE, D2), jnp.float32)

    ya, xr = _mm1(x, W1l.T, W1r.T, b1l[None, :])
    acc = _make_scatter(D1, chunks_per_tile)(ya, src_p, dst_p, zeros1)
    w2cat = jnp.concatenate([W2l.T, W2r.T], axis=1)
    z1, zr, rcnt = _mid(acc, ya, xr, gamma[None, :], beta[None, :],
                        w2cat, b2l[None, :])
    acc2 = _make_scatter(D2, chunks_per_tile)(z1, src_p, dst_p, zeros2)
    return _fin(acc2, z1, rcnt, zr)


# speedup vs baseline: 11.0336x; 1.0531x over previous
import functools

import jax
import jax.numpy as jnp
from jax import lax
from jax.experimental import pallas as pl
from jax.experimental.pallas import tpu as pltpu
from jax.experimental.pallas import tpu_sc as plsc

N = 10000
ACC_N = 10240
IN_DIM = 256
HID = 64
OUT = 16
D1 = HID + 16
D2 = OUT

NUM_CORES = 2
NUM_TILES = 16
NUM_WORKERS = NUM_CORES * NUM_TILES
CHUNK = 128
ROWS_PER_TILE = ACC_N // NUM_TILES


def _mm1_body(x_ref, wl_ref, wr_ref, b_ref, ya_ref, xr_ref):
    x = x_ref[...]
    xl = jnp.dot(x, wl_ref[...], preferred_element_type=jnp.float32,
                 precision=lax.Precision.HIGHEST)
    blk = xl.shape[0]
    ya_ref[...] = jnp.concatenate(
        [xl, jnp.ones((blk, D1 - HID), jnp.float32)], axis=1)
    xr_ref[...] = jnp.dot(x, wr_ref[...], preferred_element_type=jnp.float32,
                          precision=lax.Precision.HIGHEST) + b_ref[...]


def _mm1(x, wlT, wrT, b1l2d):
    blk = 2000
    return pl.pallas_call(
        _mm1_body,
        grid=(N // blk,),
        in_specs=[
            pl.BlockSpec((blk, IN_DIM), lambda i: (i, 0)),
            pl.BlockSpec((IN_DIM, HID), lambda i: (0, 0)),
            pl.BlockSpec((IN_DIM, HID), lambda i: (0, 0)),
            pl.BlockSpec((1, HID), lambda i: (0, 0)),
        ],
        out_specs=[
            pl.BlockSpec((blk, D1), lambda i: (i, 0)),
            pl.BlockSpec((blk, HID), lambda i: (i, 0)),
        ],
        out_shape=[
            jax.ShapeDtypeStruct((N, D1), jnp.float32),
            jax.ShapeDtypeStruct((N, HID), jnp.float32),
        ],
    )(x, wlT, wrT, b1l2d)


NBUF = 8

CPT0 = 56
CPT1 = 24


def _make_scatter(d, cpt0, cpt1):
    mesh = plsc.VectorSubcoreMesh(core_axis_name="c", subcore_axis_name="s")
    nbuf = 4 if d > 32 else NBUF

    @functools.partial(
        pl.kernel,
        mesh=mesh,
        compiler_params=pltpu.CompilerParams(use_tc_tiling_on_sc=False),
        out_type=jax.ShapeDtypeStruct((NUM_CORES * ACC_N, d), jnp.float32),
        scratch_types=[
            pltpu.VMEM((nbuf, CHUNK), jnp.int32),
            pltpu.VMEM((nbuf, CHUNK), jnp.int32),
            pltpu.VMEM((nbuf, CHUNK, d), jnp.float32),
            pltpu.VMEM_SHARED((ACC_N, d), jnp.float32),
            pltpu.SemaphoreType.DMA,
            pltpu.SemaphoreType.DMA,
        ],
    )
    def scatter(table_hbm, src_hbm, dst_hbm, zeros_hbm, out_hbm,
                src_idx, dst_idx, rows, acc_sh, gsem, ssem):
        cid = lax.axis_index("c")
        sid = lax.axis_index("s")
        cpt_c = jnp.where(cid == 0, cpt0, cpt1)
        base_chunk = cid * NUM_TILES * cpt0 + sid * cpt_c

        pltpu.sync_copy(zeros_hbm, acc_sh.at[pl.ds(sid * ROWS_PER_TILE,
                                                   ROWS_PER_TILE)])
        plsc.subcore_barrier()

        def outer(g, carry):
            cb = base_chunk + g * nbuf
            pltpu.sync_copy(src_hbm.at[pl.ds(cb, nbuf)], src_idx)
            pltpu.sync_copy(dst_hbm.at[pl.ds(cb, nbuf)], dst_idx)
            gcps = [
                pltpu.async_copy(table_hbm.at[src_idx.at[b]],
                                 rows.at[b], gsem)
                for b in range(nbuf)
            ]
            scps = []
            for b in range(nbuf):
                gcps[b].wait()
                scps.append(
                    pltpu.async_copy(rows.at[b],
                                     acc_sh.at[dst_idx.at[b]],
                                     ssem, add=True))
            for cp in scps:
                cp.wait()
            return carry

        lax.fori_loop(0, cpt_c // nbuf, outer, 0)
        plsc.subcore_barrier()

        pltpu.sync_copy(
            acc_sh.at[pl.ds(sid * ROWS_PER_TILE, ROWS_PER_TILE)],
            out_hbm.at[pl.ds(cid * ACC_N + sid * ROWS_PER_TILE,
                             ROWS_PER_TILE)])

    return scatter


def _mid_body(acc_ref, ya_ref, xr_ref, g_ref, b_ref, w2_ref, b2_ref,
              z1_ref, zr_ref, rcnt_ref):
    a = acc_ref[...]
    s = a[:N] + a[ACC_N:ACC_N + N]
    cnt = s[:, HID:HID + 1] + 1.0
    rcnt = 1.0 / cnt
    y1 = ya_ref[...][:, :HID]
    h = (s[:, :HID] + y1) * rcnt + xr_ref[...]
    mean = jnp.mean(h, axis=0, keepdims=True)
    var = jnp.mean((h - mean) ** 2, axis=0, keepdims=True)
    hn = (h - mean) * lax.rsqrt(var + 1e-5) * g_ref[...] + b_ref[...]
    hn = jnp.maximum(hn, 0.0)
    z = jnp.dot(hn, w2_ref[...], preferred_element_type=jnp.float32,
                precision=lax.Precision.HIGHEST)
    z1_ref[...] = z[:, :OUT]
    zr_ref[...] = z[:, OUT:] + b2_ref[...]
    rcnt_ref[...] = jnp.broadcast_to(rcnt, (N, OUT))


def _mid(acc, ya, xr, gamma2d, beta2d, w2cat, b2l2d):
    return pl.pallas_call(
        _mid_body,
        out_shape=[
            jax.ShapeDtypeStruct((N, OUT), jnp.float32),
            jax.ShapeDtypeStruct((N, OUT), jnp.float32),
            jax.ShapeDtypeStruct((N, OUT), jnp.float32),
        ],
    )(acc, ya, xr, gamma2d, beta2d, w2cat, b2l2d)


def _fin_body(acc2_ref, z1_ref, rcnt_ref, zr_ref, out_ref):
    a = acc2_ref[...]
    s = a[:N] + a[ACC_N:ACC_N + N]
    out_ref[...] = (s + z1_ref[...]) * rcnt_ref[...] + zr_ref[...]


def _fin(acc2, z1, rcnt, zr):
    return pl.pallas_call(
        _fin_body,
        out_shape=jax.ShapeDtypeStruct((N, OUT), jnp.float32),
    )(acc2, z1, rcnt, zr)


def kernel(x, edge_index, W1l, b1l, W1r, gamma, beta, W2l, b2l, W2r):
    e = edge_index.shape[1]
    epad = NUM_TILES * (CPT0 + CPT1) * CHUNK
    assert epad >= e and CPT0 % 8 == 0 and CPT1 % 8 == 0

    src_p = jnp.concatenate(
        [edge_index[0], jnp.zeros((epad - e,), jnp.int32)]
    ).reshape(epad // CHUNK, CHUNK)
    dst_p = jnp.concatenate(
        [edge_index[1], jnp.full((epad - e,), N, jnp.int32)]
    ).reshape(epad // CHUNK, CHUNK)
    zeros1 = jnp.zeros((ROWS_PER_TILE, D1), jnp.float32)
    zeros2 = jnp.zeros((ROWS_PER_TILE, D2), jnp.float32)

    ya, xr = _mm1(x, W1l.T, W1r.T, b1l[None, :])
    acc = _make_scatter(D1, CPT0, CPT1)(ya, src_p, dst_p, zeros1)
    w2cat = jnp.concatenate([W2l.T, W2r.T], axis=1)
    z1, zr, rcnt = _mid(acc, ya, xr, gamma[None, :], beta[None, :],
                        w2cat, b2l[None, :])
    acc2 = _make_scatter(D2, CPT0, CPT1)(z1, src_p, dst_p, zeros2)
    return _fin(acc2, z1, rcnt, zr)


# speedup vs baseline: 12.0527x; 1.0924x over previous
import functools

import jax
import jax.numpy as jnp
from jax import lax
from jax.experimental import pallas as pl
from jax.experimental.pallas import tpu as pltpu
from jax.experimental.pallas import tpu_sc as plsc

N = 10000
ACC_N = 10240
IN_DIM = 256
HID = 64
OUT = 16
D1 = HID + 16
D2 = OUT

NUM_CORES = 2
NUM_TILES = 16
NUM_WORKERS = NUM_CORES * NUM_TILES
CHUNK = 128
ROWS_PER_TILE = ACC_N // NUM_TILES


def _mm1_body(x_ref, wl_ref, wr_ref, b_ref, ya_ref, xr_ref):
    x = x_ref[...]
    xl = jnp.dot(x, wl_ref[...], preferred_element_type=jnp.float32,
                 precision=lax.Precision.HIGHEST)
    blk = xl.shape[0]
    ya_ref[...] = jnp.concatenate(
        [xl, jnp.ones((blk, D1 - HID), jnp.float32)], axis=1)
    xr_ref[...] = jnp.dot(x, wr_ref[...], preferred_element_type=jnp.float32,
                          precision=lax.Precision.HIGHEST) + b_ref[...]


def _mm1(x, wlT, wrT, b1l2d):
    blk = 2000
    return pl.pallas_call(
        _mm1_body,
        grid=(N // blk,),
        in_specs=[
            pl.BlockSpec((blk, IN_DIM), lambda i: (i, 0)),
            pl.BlockSpec((IN_DIM, HID), lambda i: (0, 0)),
            pl.BlockSpec((IN_DIM, HID), lambda i: (0, 0)),
            pl.BlockSpec((1, HID), lambda i: (0, 0)),
        ],
        out_specs=[
            pl.BlockSpec((blk, D1), lambda i: (i, 0)),
            pl.BlockSpec((blk, HID), lambda i: (i, 0)),
        ],
        out_shape=[
            jax.ShapeDtypeStruct((N, D1), jnp.float32),
            jax.ShapeDtypeStruct((N, HID), jnp.float32),
        ],
    )(x, wlT, wrT, b1l2d)


NBUF = 8

CPT0 = 72
CPT1 = 8


def _make_scatter(d, cpt0, cpt1):
    mesh = plsc.VectorSubcoreMesh(core_axis_name="c", subcore_axis_name="s")
    nbuf = 4 if d > 32 else NBUF

    @functools.partial(
        pl.kernel,
        mesh=mesh,
        compiler_params=pltpu.CompilerParams(use_tc_tiling_on_sc=False),
        out_type=jax.ShapeDtypeStruct((NUM_CORES * ACC_N, d), jnp.float32),
        scratch_types=[
            pltpu.VMEM((nbuf, CHUNK), jnp.int32),
            pltpu.VMEM((nbuf, CHUNK), jnp.int32),
            pltpu.VMEM((nbuf, CHUNK, d), jnp.float32),
            pltpu.VMEM_SHARED((ACC_N, d), jnp.float32),
            pltpu.SemaphoreType.DMA,
            pltpu.SemaphoreType.DMA,
        ],
    )
    def scatter(table_hbm, src_hbm, dst_hbm, zeros_hbm, out_hbm,
                src_idx, dst_idx, rows, acc_sh, gsem, ssem):
        cid = lax.axis_index("c")
        sid = lax.axis_index("s")
        cpt_c = jnp.where(cid == 0, cpt0, cpt1)
        base_chunk = cid * NUM_TILES * cpt0 + sid * cpt_c

        pltpu.sync_copy(zeros_hbm, acc_sh.at[pl.ds(sid * ROWS_PER_TILE,
                                                   ROWS_PER_TILE)])
        plsc.subcore_barrier()

        def outer(g, carry):
            cb = base_chunk + g * nbuf
            pltpu.sync_copy(src_hbm.at[pl.ds(cb, nbuf)], src_idx)
            pltpu.sync_copy(dst_hbm.at[pl.ds(cb, nbuf)], dst_idx)
            gcps = [
                pltpu.async_copy(table_hbm.at[src_idx.at[b]],
                                 rows.at[b], gsem)
                for b in range(nbuf)
            ]
            scps = []
            for b in range(nbuf):
                gcps[b].wait()
                scps.append(
                    pltpu.async_copy(rows.at[b],
                                     acc_sh.at[dst_idx.at[b]],
                                     ssem, add=True))
            for cp in scps:
                cp.wait()
            return carry

        lax.fori_loop(0, cpt_c // nbuf, outer, 0)
        plsc.subcore_barrier()

        pltpu.sync_copy(
            acc_sh.at[pl.ds(sid * ROWS_PER_TILE, ROWS_PER_TILE)],
            out_hbm.at[pl.ds(cid * ACC_N + sid * ROWS_PER_TILE,
                             ROWS_PER_TILE)])

    return scatter


def _mid_body(acc_ref, ya_ref, xr_ref, g_ref, b_ref, w2_ref, b2_ref,
              z1_ref, zr_ref, rcnt_ref):
    a = acc_ref[...]
    s = a[:N] + a[ACC_N:ACC_N + N]
    cnt = s[:, HID:HID + 1] + 1.0
    rcnt = 1.0 / cnt
    y1 = ya_ref[...][:, :HID]
    h = (s[:, :HID] + y1) * rcnt + xr_ref[...]
    mean = jnp.mean(h, axis=0, keepdims=True)
    var = jnp.mean((h - mean) ** 2, axis=0, keepdims=True)
    hn = (h - mean) * lax.rsqrt(var + 1e-5) * g_ref[...] + b_ref[...]
    hn = jnp.maximum(hn, 0.0)
    z = jnp.dot(hn, w2_ref[...], preferred_element_type=jnp.float32,
                precision=lax.Precision.HIGHEST)
    z1_ref[...] = z[:, :OUT]
    zr_ref[...] = z[:, OUT:] + b2_ref[...]
    rcnt_ref[...] = jnp.broadcast_to(rcnt, (N, OUT))


def _mid(acc, ya, xr, gamma2d, beta2d, w2cat, b2l2d):
    return pl.pallas_call(
        _mid_body,
        out_shape=[
            jax.ShapeDtypeStruct((N, OUT), jnp.float32),
            jax.ShapeDtypeStruct((N, OUT), jnp.float32),
            jax.ShapeDtypeStruct((N, OUT), jnp.float32),
        ],
    )(acc, ya, xr, gamma2d, beta2d, w2cat, b2l2d)


def _fin_body(acc2_ref, z1_ref, rcnt_ref, zr_ref, out_ref):
    a = acc2_ref[...]
    s = a[:N] + a[ACC_N:ACC_N + N]
    out_ref[...] = (s + z1_ref[...]) * rcnt_ref[...] + zr_ref[...]


def _fin(acc2, z1, rcnt, zr):
    return pl.pallas_call(
        _fin_body,
        out_shape=jax.ShapeDtypeStruct((N, OUT), jnp.float32),
    )(acc2, z1, rcnt, zr)


def kernel(x, edge_index, W1l, b1l, W1r, gamma, beta, W2l, b2l, W2r):
    e = edge_index.shape[1]
    epad = NUM_TILES * (CPT0 + CPT1) * CHUNK
    assert epad >= e and CPT0 % 8 == 0 and CPT1 % 8 == 0

    src_p = jnp.concatenate(
        [edge_index[0], jnp.zeros((epad - e,), jnp.int32)]
    ).reshape(epad // CHUNK, CHUNK)
    dst_p = jnp.concatenate(
        [edge_index[1], jnp.full((epad - e,), N, jnp.int32)]
    ).reshape(epad // CHUNK, CHUNK)
    zeros1 = jnp.zeros((ROWS_PER_TILE, D1), jnp.float32)
    zeros2 = jnp.zeros((ROWS_PER_TILE, D2), jnp.float32)

    ya, xr = _mm1(x, W1l.T, W1r.T, b1l[None, :])
    acc = _make_scatter(D1, CPT0, CPT1)(ya, src_p, dst_p, zeros1)
    w2cat = jnp.concatenate([W2l.T, W2r.T], axis=1)
    z1, zr, rcnt = _mid(acc, ya, xr, gamma[None, :], beta[None, :],
                        w2cat, b2l[None, :])
    acc2 = _make_scatter(D2, CPT0, CPT1)(z1, src_p, dst_p, zeros2)
    return _fin(acc2, z1, rcnt, zr)
